# Initial kernel scaffold; baseline (speedup 1.0000x reference)
import jax
import jax.numpy as jnp
from jax.experimental import pallas as pl


def kernel(h_2, W_enc, b_enc):
    raise NotImplementedError("write your pallas kernel here")


# speedup vs baseline: 23.8794x; 23.8794x over previous
import functools

import jax
import jax.numpy as jnp
from jax.experimental import pallas as pl

_LEVELS = (3072, 6144, 12288, 24576)
_TOPK = (32, 32, 64, 128)
_GAMMA = 1.0
_BETA = 1.0


def _segments(levels, topk):
    starts = (0,) + tuple(levels[:-1])
    return tuple(zip(starts, levels, topk))


def _kth_bits(bits, k):
    r = bits.shape[0]
    lo = jnp.zeros((r, 1), jnp.int32)
    hi = jnp.full((r, 1), 0x7F800000, jnp.int32)

    def body(_, carry):
        lo, hi = carry
        mid = lo + ((hi - lo) >> 1)
        cnt = jnp.sum((bits >= mid).astype(jnp.int32), axis=1, keepdims=True)
        ge = cnt >= k
        return jnp.where(ge, mid, lo), jnp.where(ge, hi, mid)

    lo, hi = jax.lax.fori_loop(0, 31, body, (lo, hi))
    return lo


def _seg_body(k, h_ref, wt_ref, b_ref, out_ref):
    zp = jax.lax.dot_general(
        h_ref[...], wt_ref[...],
        dimension_numbers=(((1,), (0,)), ((), ())),
        preferred_element_type=jnp.float32,
    ) + b_ref[...]
    z = jnp.where(zp > _GAMMA, zp + _BETA, jnp.maximum(zp, 0.0))
    bits = jax.lax.bitcast_convert_type(z, jnp.int32)
    th = _kth_bits(bits, k)
    out_ref[...] = jnp.where(bits >= th, z, 0.0)


def _seg_call(h_2, w_t, b_2d, start, width, k, row_tile):
    n_rows, d_in = h_2.shape
    grid = (n_rows // row_tile,)
    blk = start // width
    return pl.pallas_call(
        functools.partial(_seg_body, k),
        grid=grid,
        in_specs=[
            pl.BlockSpec((row_tile, d_in), lambda i: (i, 0)),
            pl.BlockSpec((d_in, width), lambda i, _b=blk: (0, _b)),
            pl.BlockSpec((1, width), lambda i, _b=blk: (0, _b)),
        ],
        out_specs=pl.BlockSpec((row_tile, width), lambda i: (i, 0)),
        out_shape=jax.ShapeDtypeStruct((n_rows, width), jnp.float32),
    )(h_2, w_t, b_2d)


def _run(levels, topk, row_tiles, h_2, w_t, b_2d):
    parts = []
    for (start, end, k), rt in zip(_segments(levels, topk), row_tiles):
        parts.append(_seg_call(h_2, w_t, b_2d, start, end - start, k, rt))
    return jnp.concatenate(parts, axis=1)


_ROW_TILES = (256, 256, 128, 64)


def kernel(h_2, W_enc, b_enc):
    w_t = W_enc.T
    b_2d = b_enc.reshape(1, -1)
    return _run(_LEVELS, _TOPK, _ROW_TILES, h_2, w_t, b_2d)


# speedup vs baseline: 26.4385x; 1.1072x over previous
import functools

import jax
import jax.numpy as jnp
from jax.experimental import pallas as pl
from jax.experimental.pallas import tpu as pltpu

_LEVELS = (3072, 6144, 12288, 24576)
_TOPK = (32, 32, 64, 128)
_GAMMA = 1.0
_BETA = 1.0


def _segments(levels, topk):
    starts = (0,) + tuple(levels[:-1])
    return tuple(zip(starts, levels, topk))


def _kth_bits(bits, k):
    r = bits.shape[0]
    lo = jnp.zeros((r, 1), jnp.int32)
    hi = jnp.full((r, 1), 0x7F800000, jnp.int32)

    def body(_, carry):
        lo, hi = carry
        mid = lo + ((hi - lo) >> 1)
        cnt = jnp.sum((bits >= mid).astype(jnp.int32), axis=1, keepdims=True)
        ge = cnt >= k
        return jnp.where(ge, mid, lo), jnp.where(ge, hi, mid)

    lo, hi = jax.lax.fori_loop(0, 31, body, (lo, hi))
    return lo


def _seg_body(k, h_ref, wt_ref, b_ref, *rest):
    out_ref = rest[-1]
    zp = jax.lax.dot_general(
        h_ref[...], wt_ref[...],
        dimension_numbers=(((1,), (0,)), ((), ())),
        preferred_element_type=jnp.float32,
    ) + b_ref[...]
    z = jnp.where(zp > _GAMMA, zp + _BETA, jnp.maximum(zp, 0.0))
    bits = jax.lax.bitcast_convert_type(z, jnp.int32)
    th = _kth_bits(bits, k)
    out_ref[...] = jnp.where(bits >= th, z, 0.0)


def _seg_call(h_2, w_t, b_2d, prev, d_lat, start, width, k, row_tile):
    n_rows, d_in = h_2.shape
    grid = (n_rows // row_tile,)
    blk = start // width
    in_specs = [
        pl.BlockSpec((row_tile, d_in), lambda i: (i, 0)),
        pl.BlockSpec((d_in, width), lambda i, _b=blk: (0, _b)),
        pl.BlockSpec((1, width), lambda i, _b=blk: (0, _b)),
    ]
    args = [h_2, w_t, b_2d]
    aliases = {}
    if prev is not None:
        in_specs.append(pl.BlockSpec(memory_space=pl.ANY))
        args.append(prev)
        aliases = {3: 0}
    return pl.pallas_call(
        functools.partial(_seg_body, k),
        grid=grid,
        in_specs=in_specs,
        out_specs=pl.BlockSpec((row_tile, width), lambda i, _b=blk: (i, _b)),
        out_shape=jax.ShapeDtypeStruct((n_rows, d_lat), jnp.float32),
        input_output_aliases=aliases,
    )(*args)


def _run(levels, topk, row_tiles, h_2, w_t, b_2d):
    d_lat = levels[-1]
    out = None
    for (start, end, k), rt in zip(_segments(levels, topk), row_tiles):
        out = _seg_call(h_2, w_t, b_2d, out, d_lat, start, end - start, k, rt)
    return out


_ROW_TILES = (256, 256, 128, 64)


def kernel(h_2, W_enc, b_enc):
    w_t = W_enc.T
    b_2d = b_enc.reshape(1, -1)
    return _run(_LEVELS, _TOPK, _ROW_TILES, h_2, w_t, b_2d)
